# baseline (speedup 1.0000x reference)
import jax
import jax.numpy as jnp
from jax.experimental import pallas as pl
from jax.experimental.pallas import tpu as pltpu

_DEPTH = 1000
_ROWS = 4096
_COLS = 26
_D_BLK = 40
_B_BLK = 256


def _body(ids_ref, out_ref):
    d0 = pl.program_id(0) * _D_BLK
    iota = jax.lax.broadcasted_iota(jnp.int32, (_D_BLK, _B_BLK), 0) + d0
    ids = ids_ref[...]
    for j in range(_COLS):
        idj = ids[j, :][None, :]
        out_ref[j] = (iota == idj).astype(jnp.float32)


def kernel(inputs):
    ids_t = jnp.transpose(inputs.astype(jnp.int32), (1, 0))
    out_t = pl.pallas_call(
        _body,
        grid=(_DEPTH // _D_BLK, _ROWS // _B_BLK),
        in_specs=[pl.BlockSpec((_COLS, _B_BLK), lambda d, b: (0, b))],
        out_specs=pl.BlockSpec((_COLS, _D_BLK, _B_BLK), lambda d, b: (0, d, b)),
        out_shape=jax.ShapeDtypeStruct((_COLS, _DEPTH, _ROWS), jnp.float32),
        compiler_params=pltpu.CompilerParams(
            dimension_semantics=("parallel", "parallel"),
        ),
    )(ids_t)
    return jnp.transpose(out_t, (2, 0, 1))


# speedup vs baseline: 2.1852x; 2.1852x over previous
import jax
import jax.numpy as jnp
from jax.experimental import pallas as pl
from jax.experimental.pallas import tpu as pltpu

_DEPTH = 1000
_ROWS = 4096
_COLS = 26
_D_BLK = 1000
_B_BLK = 256


def _body(ids_ref, out_ref):
    d0 = pl.program_id(0) * _D_BLK
    iota = jax.lax.broadcasted_iota(jnp.int32, (_D_BLK, _B_BLK), 0) + d0
    ids = ids_ref[...]
    for j in range(_COLS):
        idj = ids[j, :][None, :]
        out_ref[j] = (iota == idj).astype(jnp.float32)


def kernel(inputs):
    ids_t = jnp.transpose(inputs.astype(jnp.int32), (1, 0))
    out_t = pl.pallas_call(
        _body,
        grid=(_DEPTH // _D_BLK, _ROWS // _B_BLK),
        in_specs=[pl.BlockSpec((_COLS, _B_BLK), lambda d, b: (0, b))],
        out_specs=pl.BlockSpec((_COLS, _D_BLK, _B_BLK), lambda d, b: (0, d, b)),
        out_shape=jax.ShapeDtypeStruct((_COLS, _DEPTH, _ROWS), jnp.float32),
        compiler_params=pltpu.CompilerParams(
            dimension_semantics=("parallel", "parallel"),
        ),
    )(ids_t)
    return jnp.transpose(out_t, (2, 0, 1))


# speedup vs baseline: 2.3023x; 1.0536x over previous
import jax
import jax.numpy as jnp
from jax.experimental import pallas as pl
from jax.experimental.pallas import tpu as pltpu

_DEPTH = 1000
_ROWS = 4096
_COLS = 26
_D_BLK = 200
_B_BLK = 256


def _body(ids_ref, out_ref):
    d0 = pl.program_id(1) * _D_BLK
    iota = jax.lax.broadcasted_iota(jnp.int32, (_D_BLK, _B_BLK), 0) + d0
    ids = ids_ref[...]
    for j in range(_COLS):
        idj = ids[j, :][None, :]
        out_ref[j] = (iota == idj).astype(jnp.float32)


def kernel(inputs):
    ids_t = jnp.transpose(inputs.astype(jnp.int32), (1, 0))
    out_t = pl.pallas_call(
        _body,
        grid=(_ROWS // _B_BLK, _DEPTH // _D_BLK),
        in_specs=[pl.BlockSpec((_COLS, _B_BLK), lambda b, d: (0, b))],
        out_specs=pl.BlockSpec((_COLS, _D_BLK, _B_BLK), lambda b, d: (0, d, b)),
        out_shape=jax.ShapeDtypeStruct((_COLS, _DEPTH, _ROWS), jnp.float32),
        compiler_params=pltpu.CompilerParams(
            dimension_semantics=("parallel", "parallel"),
        ),
    )(ids_t)
    return jnp.transpose(out_t, (2, 0, 1))
